# baseline (speedup 1.0000x reference)
import functools

import jax
import jax.numpy as jnp
from jax import lax
from jax.experimental import pallas as pl
from jax.experimental.pallas import tpu as pltpu
from jax.experimental.pallas import tpu_sc as plsc

N_OUT = 32768
N_IN = 2048
BS = 512
L = 16
NW = 32
BSC = 32
NHALF = 2
RB = 1024
NB = N_OUT // RB
BPH = NB // NHALF
CH = 4096
NBOUNDS = 64


def _sc_sparse_matmul(inp2, packed, wch, bounds, bkg, rest10):
    nch = packed.shape[0]
    mesh = plsc.VectorSubcoreMesh(core_axis_name="c", subcore_axis_name="s")

    @functools.partial(
        pl.kernel,
        out_type=jax.ShapeDtypeStruct((BS, N_OUT), jnp.float32),
        mesh=mesh,
        compiler_params=pltpu.CompilerParams(
            needs_layout_passes=False,
            use_tc_tiling_on_sc=False,
        ),
        scratch_types=[
            pltpu.VMEM((BSC, N_IN), jnp.float32),
            pltpu.VMEM((BSC, RB), jnp.float32),
            pltpu.VMEM((2, CH), jnp.int32),
            pltpu.VMEM((CH,), jnp.float32),
            pltpu.VMEM((NBOUNDS,), jnp.int32),
            pltpu.VMEM((RB,), jnp.float32),
            pltpu.VMEM((BSC, L), jnp.float32),
        ],
    )
    def body(inp_hbm, packed_hbm, wch_hbm, bounds_hbm, bkg_hbm, rest_hbm,
             out_hbm, inp_v, acc_v, chunk_v, chunkw_v, bounds_v, bkg_v,
             restm_v):
        wid = lax.axis_index("s") * 2 + lax.axis_index("c")
        half = wid % NHALF
        bs0 = (wid // NHALF) * BSC

        pltpu.sync_copy(inp_hbm.at[pl.ds(bs0, BSC), :], inp_v)
        pltpu.sync_copy(rest_hbm.at[pl.ds(bs0, BSC), :], restm_v)
        pltpu.sync_copy(bounds_hbm, bounds_v)

        iota = lax.iota(jnp.int32, L)
        jsplat = [jnp.full((L,), j, jnp.int32) for j in range(BSC)]

        def pick(g):
            return bounds_v[pl.ds(g, L)][0]

        def block_body(b, _):
            g = half * BPH + b
            base = g * RB
            s = pick(g)
            e = pick(g + 1)

            pltpu.sync_copy(bkg_hbm.at[pl.ds(base, RB)], bkg_v)

            def init_body(r, _):
                bk = bkg_v[pl.ds(r * L, L)]
                for j in range(BSC):
                    acc_v[j, pl.ds(r * L, L)] = bk * restm_v[j, :]
                return 0

            lax.fori_loop(0, RB // L, init_body, 0)

            t0 = s // CH
            t1 = lax.max(t0, (e - 1) // CH)
            ntc = jnp.where(e > s, t1 - t0 + 1, 0)

            def chunk_body(ci, _):
                t = t0 + ci
                tbase = t * CH
                pltpu.sync_copy(packed_hbm.at[t], chunk_v)
                pltpu.sync_copy(wch_hbm.at[t], chunkw_v)
                klo = lax.max(s - tbase, 0) // L
                khi = (lax.min(e - tbase, CH) + (L - 1)) // L

                def group_body(k, _):
                    off = k * L
                    rows16 = chunk_v[0, pl.ds(off, L)]
                    cols16 = chunk_v[1, pl.ds(off, L)]
                    w16 = chunkw_v[pl.ds(off, L)]
                    kg = tbase + off + iota
                    valid = (kg >= s) & (kg < e)
                    rloc = rows16 - base
                    for j in range(BSC):
                        g16 = plsc.load_gather(inp_v, [jsplat[j], cols16])
                        plsc.addupdate_scatter(acc_v, [jsplat[j], rloc],
                                               g16 * w16, mask=valid)
                    return 0

                lax.fori_loop(klo, khi, group_body, 0)
                return 0

            lax.fori_loop(0, ntc, chunk_body, 0)

            pltpu.sync_copy(acc_v,
                            out_hbm.at[pl.ds(bs0, BSC), pl.ds(base, RB)])
            return 0

        lax.fori_loop(0, BPH, block_body, 0)

    return body(inp2, packed, wch, bounds, bkg, rest10)


def kernel(inp, indices, weights, bkg_weights):
    b, s, f = inp.shape
    inp2 = inp.reshape(b * s, f).astype(jnp.float32)
    rows = indices[:, 0].astype(jnp.int32)
    cols = indices[:, 1].astype(jnp.int32)
    w32 = weights.astype(jnp.float32)

    nnz = rows.shape[0]
    pad = (-nnz) % CH
    if pad:
        rows_p = jnp.pad(rows, (0, pad), constant_values=N_OUT - 1)
        cols_p = jnp.pad(cols, (0, pad))
        w_p = jnp.pad(w32, (0, pad))
    else:
        rows_p, cols_p, w_p = rows, cols, w32
    packed = jnp.stack([rows_p, cols_p], axis=1)
    packed = packed.reshape(-1, CH, 2).transpose(0, 2, 1)
    wch = w_p.reshape(-1, CH)

    edges = jnp.arange(0, N_OUT + 1, RB, dtype=jnp.int32)
    bounds = jnp.searchsorted(rows, edges, side="left").astype(jnp.int32)
    bounds = jnp.pad(bounds, (0, NBOUNDS - bounds.shape[0]))

    kn = jax.random.key(42)
    rest = jnp.sum((jax.random.uniform(kn, (b, s, 10)) < 0.1)
                   .astype(jnp.float32), -1).reshape(b * s)
    rest10 = jnp.broadcast_to((rest / 10.0)[:, None], (b * s, L))

    out2 = _sc_sparse_matmul(inp2, packed, wch, bounds,
                             bkg_weights.astype(jnp.float32), rest10)
    return out2.reshape(b, s, N_OUT)


# speedup vs baseline: 2.2570x; 2.2570x over previous
import functools

import jax
import jax.numpy as jnp
from jax import lax
from jax.experimental import pallas as pl
from jax.experimental.pallas import tpu as pltpu
from jax.experimental.pallas import tpu_sc as plsc

N_OUT = 32768
N_IN = 2048
BS = 512
L = 16
NW = 32
BSC = 32
NHALF = 2
RB = 1024
NB = N_OUT // RB
BPH = NB // NHALF
CH = 512
S = CH // L
NG = CH // L
NBOUNDS = 64


def _sc_sparse_matmul(inp2, packed, wch, bounds, bkg, rest10):
    nch = packed.shape[0]
    mesh = plsc.VectorSubcoreMesh(core_axis_name="c", subcore_axis_name="s")

    @functools.partial(
        pl.kernel,
        out_type=jax.ShapeDtypeStruct((BS, N_OUT), jnp.float32),
        mesh=mesh,
        compiler_params=pltpu.CompilerParams(
            needs_layout_passes=False,
            use_tc_tiling_on_sc=False,
        ),
        scratch_types=[
            pltpu.VMEM((BSC, N_IN), jnp.float32),
            pltpu.VMEM((BSC, RB), jnp.float32),
            pltpu.VMEM((2, CH), jnp.int32),
            pltpu.VMEM((CH,), jnp.float32),
            pltpu.VMEM((NBOUNDS,), jnp.int32),
            pltpu.VMEM((RB,), jnp.float32),
            pltpu.VMEM((BSC, L), jnp.float32),
        ],
    )
    def body(inp_hbm, packed_hbm, wch_hbm, bounds_hbm, bkg_hbm, rest_hbm,
             out_hbm, inp_v, acc_v, chunk_v, chunkw_v, bounds_v, bkg_v,
             restm_v):
        wid = lax.axis_index("s") * 2 + lax.axis_index("c")
        half = wid % NHALF
        bs0 = (wid // NHALF) * BSC

        pltpu.sync_copy(inp_hbm.at[pl.ds(bs0, BSC), :], inp_v)
        pltpu.sync_copy(rest_hbm.at[pl.ds(bs0, BSC), :], restm_v)
        pltpu.sync_copy(bounds_hbm, bounds_v)

        iota = lax.iota(jnp.int32, L)
        jsplat = [jnp.full((L,), j, jnp.int32) for j in range(BSC)]

        def pick(g):
            return bounds_v[pl.ds(g, L)][0]

        def block_body(b, _):
            g = half * BPH + b
            base = g * RB
            s = pick(g)
            e = pick(g + 1)

            pltpu.sync_copy(bkg_hbm.at[pl.ds(base, RB)], bkg_v)

            def init_body(r, _):
                bk = bkg_v[pl.ds(r * L, L)]
                for j in range(BSC):
                    acc_v[j, pl.ds(r * L, L)] = bk * restm_v[j, :]
                return 0

            lax.fori_loop(0, RB // L, init_body, 0)

            t0 = s // CH
            t1 = lax.max(t0, (e - 1) // CH)
            ntc = jnp.where(e > s, t1 - t0 + 1, 0)

            def chunk_body(ci, _):
                t = t0 + ci
                tbase = t * CH
                pltpu.sync_copy(packed_hbm.at[t], chunk_v)
                pltpu.sync_copy(wch_hbm.at[t], chunkw_v)

                @plsc.parallel_loop(0, NG, 1, unroll=2)
                def group_body(k):
                    off = k * L
                    rows16 = chunk_v[0, pl.ds(off, L)]
                    cols16 = chunk_v[1, pl.ds(off, L)]
                    w16 = chunkw_v[pl.ds(off, L)]
                    kg = tbase + iota * S + k
                    valid = (kg >= s) & (kg < e)
                    rloc = rows16 - base
                    for j in range(BSC):
                        g16 = plsc.load_gather(inp_v, [jsplat[j], cols16])
                        plsc.addupdate_scatter(acc_v, [jsplat[j], rloc],
                                               g16 * w16, mask=valid)

                return 0

            lax.fori_loop(0, ntc, chunk_body, 0)

            pltpu.sync_copy(acc_v,
                            out_hbm.at[pl.ds(bs0, BSC), pl.ds(base, RB)])
            return 0

        lax.fori_loop(0, BPH, block_body, 0)

    return body(inp2, packed, wch, bounds, bkg, rest10)


def kernel(inp, indices, weights, bkg_weights):
    b, s, f = inp.shape
    inp2 = inp.reshape(b * s, f).astype(jnp.float32)
    rows = indices[:, 0].astype(jnp.int32)
    cols = indices[:, 1].astype(jnp.int32)
    w32 = weights.astype(jnp.float32)

    nnz = rows.shape[0]
    pad = (-nnz) % CH
    if pad:
        rows_p = jnp.pad(rows, (0, pad), constant_values=N_OUT - 1)
        cols_p = jnp.pad(cols, (0, pad))
        w_p = jnp.pad(w32, (0, pad))
    else:
        rows_p, cols_p, w_p = rows, cols, w32
    lane = jnp.arange(CH)
    perm = (lane % L) * S + lane // L
    packed = jnp.stack([rows_p, cols_p], axis=1)
    packed = packed.reshape(-1, CH, 2).transpose(0, 2, 1)
    packed = packed[:, :, perm]
    wch = w_p.reshape(-1, CH)[:, perm]

    edges = jnp.arange(0, N_OUT + 1, RB, dtype=jnp.int32)
    bounds = jnp.searchsorted(rows, edges, side="left").astype(jnp.int32)
    bounds = jnp.pad(bounds, (0, NBOUNDS - bounds.shape[0]))

    kn = jax.random.key(42)
    rest = jnp.sum((jax.random.uniform(kn, (b, s, 10)) < 0.1)
                   .astype(jnp.float32), -1).reshape(b * s)
    rest10 = jnp.broadcast_to((rest / 10.0)[:, None], (b * s, L))

    out2 = _sc_sparse_matmul(inp2, packed, wch, bounds,
                             bkg_weights.astype(jnp.float32), rest10)
    return out2.reshape(b, s, N_OUT)


# speedup vs baseline: 3.3313x; 1.4760x over previous
import functools

import jax
import jax.numpy as jnp
from jax import lax
from jax.experimental import pallas as pl
from jax.experimental.pallas import tpu as pltpu
from jax.experimental.pallas import tpu_sc as plsc

N_OUT = 32768
N_IN = 2048
BS = 512
L = 16
NW = 32
BSC = 32
NHALF = 2
RB = 1024
NB = N_OUT // RB
BPH = NB // NHALF
CH = 512
S = CH // L
NG = CH // L
NBOUNDS = 64


def _sc_sparse_matmul(inp2, packed, wch, bounds, bkg, rest10):
    nch = packed.shape[0]
    mesh = plsc.VectorSubcoreMesh(core_axis_name="c", subcore_axis_name="s")

    @functools.partial(
        pl.kernel,
        out_type=jax.ShapeDtypeStruct((BS, N_OUT), jnp.float32),
        mesh=mesh,
        compiler_params=pltpu.CompilerParams(
            needs_layout_passes=False,
            use_tc_tiling_on_sc=False,
        ),
        scratch_types=[
            pltpu.VMEM((BSC, N_IN), jnp.float32),
            pltpu.VMEM((BSC, RB), jnp.float32),
            pltpu.VMEM((2, CH), jnp.int32),
            pltpu.VMEM((CH,), jnp.float32),
            pltpu.VMEM((NBOUNDS,), jnp.int32),
            pltpu.VMEM((RB,), jnp.float32),
            pltpu.VMEM((BSC, L), jnp.float32),
        ],
    )
    def body(inp_hbm, packed_hbm, wch_hbm, bounds_hbm, bkg_hbm, rest_hbm,
             out_hbm, inp_v, acc_v, chunk_v, chunkw_v, bounds_v, bkg_v,
             restm_v):
        wid = lax.axis_index("s") * 2 + lax.axis_index("c")
        half = wid % NHALF
        bs0 = (wid // NHALF) * BSC

        pltpu.sync_copy(inp_hbm.at[pl.ds(bs0, BSC), :], inp_v)
        pltpu.sync_copy(rest_hbm.at[pl.ds(bs0, BSC), :], restm_v)
        pltpu.sync_copy(bounds_hbm, bounds_v)

        iota = lax.iota(jnp.int32, L)
        jsplat = [jnp.full((L,), j, jnp.int32) for j in range(BSC)]

        def pick(g):
            return bounds_v[pl.ds(g, L)][0]

        def block_body(b, _):
            g = half * BPH + b
            base = g * RB
            s = pick(g)
            e = pick(g + 1)

            pltpu.sync_copy(bkg_hbm.at[pl.ds(base, RB)], bkg_v)

            def init_body(r, _):
                bk = bkg_v[pl.ds(r * L, L)]
                for j in range(BSC):
                    acc_v[j, pl.ds(r * L, L)] = bk * restm_v[j, :]
                return 0

            lax.fori_loop(0, RB // L, init_body, 0)

            t0 = s // CH

            def process_chunk(t, masked):
                tbase = t * CH
                pltpu.sync_copy(packed_hbm.at[t], chunk_v)
                pltpu.sync_copy(wch_hbm.at[t], chunkw_v)

                @plsc.parallel_loop(0, NG, 1, unroll=4)
                def group_body(k):
                    off = k * L
                    rows16 = chunk_v[0, pl.ds(off, L)]
                    cols16 = chunk_v[1, pl.ds(off, L)]
                    w16 = chunkw_v[pl.ds(off, L)]
                    rloc = rows16 - base
                    if masked:
                        kg = tbase + iota * S + k
                        valid = (kg >= s) & (kg < e)
                    else:
                        valid = None
                    for j in range(BSC):
                        g16 = plsc.load_gather(inp_v.at[j], [cols16])
                        plsc.addupdate_scatter(acc_v.at[j], [rloc],
                                               g16 * w16, mask=valid)

            @pl.when(e > s)
            def _():
                t1b = (e - 1) // CH
                process_chunk(t0, True)

                def interior(t, _):
                    process_chunk(t, False)
                    return 0

                lax.fori_loop(t0 + 1, t1b, interior, 0)

                @pl.when(t1b > t0)
                def _():
                    process_chunk(t1b, True)

            pltpu.sync_copy(acc_v,
                            out_hbm.at[pl.ds(bs0, BSC), pl.ds(base, RB)])
            return 0

        lax.fori_loop(0, BPH, block_body, 0)

    return body(inp2, packed, wch, bounds, bkg, rest10)


def kernel(inp, indices, weights, bkg_weights):
    b, s, f = inp.shape
    inp2 = inp.reshape(b * s, f).astype(jnp.float32)
    rows = indices[:, 0].astype(jnp.int32)
    cols = indices[:, 1].astype(jnp.int32)
    w32 = weights.astype(jnp.float32)

    nnz = rows.shape[0]
    pad = (-nnz) % CH
    if pad:
        rows_p = jnp.pad(rows, (0, pad), constant_values=N_OUT - 1)
        cols_p = jnp.pad(cols, (0, pad))
        w_p = jnp.pad(w32, (0, pad))
    else:
        rows_p, cols_p, w_p = rows, cols, w32
    lane = jnp.arange(CH)
    perm = (lane % L) * S + lane // L
    packed = jnp.stack([rows_p, cols_p], axis=1)
    packed = packed.reshape(-1, CH, 2).transpose(0, 2, 1)
    packed = packed[:, :, perm]
    wch = w_p.reshape(-1, CH)[:, perm]

    edges = jnp.arange(0, N_OUT + 1, RB, dtype=jnp.int32)
    bounds = jnp.searchsorted(rows, edges, side="left").astype(jnp.int32)
    bounds = jnp.pad(bounds, (0, NBOUNDS - bounds.shape[0]))

    kn = jax.random.key(42)
    rest = jnp.sum((jax.random.uniform(kn, (b, s, 10)) < 0.1)
                   .astype(jnp.float32), -1).reshape(b * s)
    rest10 = jnp.broadcast_to((rest / 10.0)[:, None], (b * s, L))

    out2 = _sc_sparse_matmul(inp2, packed, wch, bounds,
                             bkg_weights.astype(jnp.float32), rest10)
    return out2.reshape(b, s, N_OUT)


# speedup vs baseline: 3.9552x; 1.1873x over previous
import functools

import jax
import jax.numpy as jnp
from jax import lax
from jax.experimental import pallas as pl
from jax.experimental.pallas import tpu as pltpu
from jax.experimental.pallas import tpu_sc as plsc

N_OUT = 32768
N_IN = 2048
BS = 512
L = 16
BSC = 32
NHALF = 2
RB = 1024
NB = N_OUT // RB
BPH = NB // NHALF
CH = 512
S = CH // L
NG = CH // L
NBOUNDS = 64


def _sc_sparse_matmul(inp2, packed, bounds, bkg, rest10):
    mesh = plsc.VectorSubcoreMesh(core_axis_name="c", subcore_axis_name="s")

    @functools.partial(
        pl.kernel,
        out_type=jax.ShapeDtypeStruct((BS, N_OUT), jnp.float32),
        mesh=mesh,
        compiler_params=pltpu.CompilerParams(
            needs_layout_passes=False,
            use_tc_tiling_on_sc=False,
        ),
        scratch_types=[
            pltpu.VMEM((BSC, N_IN), jnp.float32),
            pltpu.VMEM((BSC, RB), jnp.float32),
            pltpu.VMEM((3, CH), jnp.float32),
            pltpu.VMEM((3, CH), jnp.float32),
            pltpu.VMEM((NBOUNDS,), jnp.int32),
            pltpu.VMEM((RB,), jnp.float32),
            pltpu.VMEM((BSC, L), jnp.float32),
            pltpu.SemaphoreType.DMA,
            pltpu.SemaphoreType.DMA,
        ],
    )
    def body(inp_hbm, packed_hbm, bounds_hbm, bkg_hbm, rest_hbm,
             out_hbm, inp_v, acc_v, chA_v, chB_v, bounds_v, bkg_v,
             restm_v, semA, semB):
        wid = lax.axis_index("s") * 2 + lax.axis_index("c")
        half = wid % NHALF
        bs0 = (wid // NHALF) * BSC

        pltpu.sync_copy(inp_hbm.at[pl.ds(bs0, BSC), :], inp_v)
        pltpu.sync_copy(rest_hbm.at[pl.ds(bs0, BSC), :], restm_v)
        pltpu.sync_copy(bounds_hbm, bounds_v)

        iota_s = lax.iota(jnp.int32, L) * S

        def pick(g):
            return bounds_v[pl.ds(g, L)][0]

        def block_body(b, _):
            g = half * BPH + b
            base = g * RB
            s = pick(g)
            e = pick(g + 1)

            pltpu.sync_copy(bkg_hbm.at[pl.ds(base, RB)], bkg_v)

            def init_body(r, _):
                bk = bkg_v[pl.ds(r * L, L)]
                for j in range(BSC):
                    acc_v[j, pl.ds(r * L, L)] = bk * restm_v[j, :]
                return 0

            lax.fori_loop(0, RB // L, init_body, 0)

            def process_chunk(buf, t):
                tbase = t * CH

                @plsc.parallel_loop(0, NG, 1, unroll=4)
                def group_body(k):
                    off = k * L
                    rows16 = buf[0, pl.ds(off, L)].astype(jnp.int32)
                    cols16 = buf[1, pl.ds(off, L)].astype(jnp.int32)
                    w16 = buf[2, pl.ds(off, L)]
                    kg = tbase + iota_s + k
                    valid = (kg >= s) & (kg < e)
                    rloc = rows16 - base
                    for j in range(BSC):
                        g16 = plsc.load_gather(inp_v.at[j], [cols16])
                        plsc.addupdate_scatter(acc_v.at[j], [rloc],
                                               g16 * w16, mask=valid)

            @pl.when(e > s)
            def _():
                t0 = s // CH
                t1 = (e - 1) // CH
                ntc = t1 - t0 + 1
                pltpu.async_copy(packed_hbm.at[t0], chA_v, semA)

                def chunk_loop(ci, _):
                    t = t0 + ci

                    @pl.when(ci % 2 == 0)
                    def _():
                        pltpu.make_async_copy(packed_hbm.at[t0], chA_v,
                                              semA).wait()

                        @pl.when(t + 1 <= t1)
                        def _():
                            pltpu.async_copy(packed_hbm.at[t + 1], chB_v,
                                             semB)

                        process_chunk(chA_v, t)

                    @pl.when(ci % 2 == 1)
                    def _():
                        pltpu.make_async_copy(packed_hbm.at[t0], chB_v,
                                              semB).wait()

                        @pl.when(t + 1 <= t1)
                        def _():
                            pltpu.async_copy(packed_hbm.at[t + 1], chA_v,
                                             semA)

                        process_chunk(chB_v, t)

                    return 0

                lax.fori_loop(0, ntc, chunk_loop, 0)

            pltpu.sync_copy(acc_v,
                            out_hbm.at[pl.ds(bs0, BSC), pl.ds(base, RB)])
            return 0

        lax.fori_loop(0, BPH, block_body, 0)

    return body(inp2, packed, bounds, bkg, rest10)


def kernel(inp, indices, weights, bkg_weights):
    b, s, f = inp.shape
    inp2 = inp.reshape(b * s, f).astype(jnp.float32)
    rows = indices[:, 0].astype(jnp.int32)
    cols = indices[:, 1].astype(jnp.int32)
    w32 = weights.astype(jnp.float32)

    nnz = rows.shape[0]
    pad = (-nnz) % CH
    if pad:
        rows_p = jnp.pad(rows, (0, pad), constant_values=N_OUT - 1)
        cols_p = jnp.pad(cols, (0, pad))
        w_p = jnp.pad(w32, (0, pad))
    else:
        rows_p, cols_p, w_p = rows, cols, w32
    lane = jnp.arange(CH)
    perm = (lane % L) * S + lane // L
    packed = jnp.stack([rows_p.astype(jnp.float32),
                        cols_p.astype(jnp.float32), w_p], axis=1)
    packed = packed.reshape(-1, CH, 3).transpose(0, 2, 1)
    packed = packed[:, :, perm]

    edges = jnp.arange(0, N_OUT + 1, RB, dtype=jnp.int32)
    bounds = jnp.searchsorted(rows, edges, side="left").astype(jnp.int32)
    bounds = jnp.pad(bounds, (0, NBOUNDS - bounds.shape[0]))

    kn = jax.random.key(42)
    rest = jnp.sum((jax.random.uniform(kn, (b, s, 10)) < 0.1)
                   .astype(jnp.float32), -1).reshape(b * s)
    rest10 = jnp.broadcast_to((rest / 10.0)[:, None], (b * s, L))

    out2 = _sc_sparse_matmul(inp2, packed, bounds,
                             bkg_weights.astype(jnp.float32), rest10)
    return out2.reshape(b, s, N_OUT)


# speedup vs baseline: 4.2160x; 1.0659x over previous
import functools

import jax
import jax.numpy as jnp
from jax import lax
from jax.experimental import pallas as pl
from jax.experimental.pallas import tpu as pltpu
from jax.experimental.pallas import tpu_sc as plsc

N_OUT = 32768
N_IN = 2048
BS = 512
L = 16
BSC = 32
NHALF = 2
RB = 1024
NB = N_OUT // RB
BPH = NB // NHALF
CH = 512
S = CH // L
NG = CH // L
NBOUNDS = 64


def _sc_sparse_matmul(inp2, packed, bounds, bkg, rest10):
    mesh = plsc.VectorSubcoreMesh(core_axis_name="c", subcore_axis_name="s")

    @functools.partial(
        pl.kernel,
        out_type=jax.ShapeDtypeStruct((BS, N_OUT), jnp.float32),
        mesh=mesh,
        compiler_params=pltpu.CompilerParams(
            needs_layout_passes=False,
            use_tc_tiling_on_sc=False,
        ),
        scratch_types=[
            pltpu.VMEM((BSC, N_IN), jnp.float32),
            pltpu.VMEM((BSC, RB), jnp.float32),
            pltpu.VMEM((3, CH), jnp.float32),
            pltpu.VMEM((3, CH), jnp.float32),
            pltpu.VMEM((NBOUNDS,), jnp.int32),
            pltpu.VMEM((RB,), jnp.float32),
            pltpu.VMEM((BSC, L), jnp.float32),
            pltpu.SemaphoreType.DMA,
            pltpu.SemaphoreType.DMA,
        ],
    )
    def body(inp_hbm, packed_hbm, bounds_hbm, bkg_hbm, rest_hbm,
             out_hbm, inp_v, acc_v, chA_v, chB_v, bounds_v, bkg_v,
             restm_v, semA, semB):
        wid = lax.axis_index("s") * 2 + lax.axis_index("c")
        half = wid % NHALF
        bs0 = (wid // NHALF) * BSC

        pltpu.sync_copy(inp_hbm.at[pl.ds(bs0, BSC), :], inp_v)
        pltpu.sync_copy(rest_hbm.at[pl.ds(bs0, BSC), :], restm_v)
        pltpu.sync_copy(bounds_hbm, bounds_v)

        iota_s = lax.iota(jnp.int32, L) * S

        def pick(g):
            return bounds_v[pl.ds(g, L)][0]

        def block_body(b, _):
            g = half * BPH + b
            base = g * RB
            s = pick(g)
            e = pick(g + 1)

            pltpu.sync_copy(bkg_hbm.at[pl.ds(base, RB)], bkg_v)

            def init_body(r, _):
                bk = bkg_v[pl.ds(r * L, L)]
                for j in range(BSC):
                    acc_v[j, pl.ds(r * L, L)] = bk * restm_v[j, :]
                return 0

            lax.fori_loop(0, RB // L, init_body, 0)

            def process_chunk(buf, t, masked):
                tbase = t * CH

                @plsc.parallel_loop(0, NG, 1, unroll=4)
                def group_body(k):
                    off = k * L
                    rloc = buf[0, pl.ds(off, L)].astype(jnp.int32)
                    cols16 = buf[1, pl.ds(off, L)].astype(jnp.int32)
                    w16 = buf[2, pl.ds(off, L)]
                    if masked:
                        kg = tbase + iota_s + k
                        valid = (kg >= s) & (kg < e)
                    else:
                        valid = None
                    for j in range(BSC):
                        g16 = plsc.load_gather(inp_v.at[j], [cols16])
                        plsc.addupdate_scatter(acc_v.at[j], [rloc],
                                               g16 * w16, mask=valid)

            @pl.when(e > s)
            def _():
                t0 = s // CH
                t1 = (e - 1) // CH
                ntc = t1 - t0 + 1
                pltpu.async_copy(packed_hbm.at[t0], chA_v, semA)

                def chunk_loop(ci, _):
                    t = t0 + ci

                    @pl.when(ci % 2 == 0)
                    def _():
                        pltpu.make_async_copy(packed_hbm.at[t0], chA_v,
                                              semA).wait()

                        @pl.when(t + 1 <= t1)
                        def _():
                            pltpu.async_copy(packed_hbm.at[t + 1], chB_v,
                                             semB)

                        bnd = (t == t0) | (t == t1)

                        @pl.when(bnd)
                        def _():
                            process_chunk(chA_v, t, True)

                        @pl.when(jnp.logical_not(bnd))
                        def _():
                            process_chunk(chA_v, t, False)

                    @pl.when(ci % 2 == 1)
                    def _():
                        pltpu.make_async_copy(packed_hbm.at[t0], chB_v,
                                              semB).wait()

                        @pl.when(t + 1 <= t1)
                        def _():
                            pltpu.async_copy(packed_hbm.at[t + 1], chA_v,
                                             semA)

                        bnd = (t == t0) | (t == t1)

                        @pl.when(bnd)
                        def _():
                            process_chunk(chB_v, t, True)

                        @pl.when(jnp.logical_not(bnd))
                        def _():
                            process_chunk(chB_v, t, False)

                    return 0

                lax.fori_loop(0, ntc, chunk_loop, 0)

            pltpu.sync_copy(acc_v,
                            out_hbm.at[pl.ds(bs0, BSC), pl.ds(base, RB)])
            return 0

        lax.fori_loop(0, BPH, block_body, 0)

    return body(inp2, packed, bounds, bkg, rest10)


def kernel(inp, indices, weights, bkg_weights):
    b, s, f = inp.shape
    inp2 = inp.reshape(b * s, f).astype(jnp.float32)
    rows = indices[:, 0].astype(jnp.int32)
    cols = indices[:, 1].astype(jnp.int32)
    w32 = weights.astype(jnp.float32)

    nnz = rows.shape[0]
    pad = (-nnz) % CH
    if pad:
        rows_p = jnp.pad(rows, (0, pad), constant_values=N_OUT - 1)
        cols_p = jnp.pad(cols, (0, pad))
        w_p = jnp.pad(w32, (0, pad))
    else:
        rows_p, cols_p, w_p = rows, cols, w32
    lane = jnp.arange(CH)
    perm = (lane % L) * S + lane // L
    packed = jnp.stack([(rows_p % RB).astype(jnp.float32),
                        cols_p.astype(jnp.float32), w_p], axis=1)
    packed = packed.reshape(-1, CH, 3).transpose(0, 2, 1)
    packed = packed[:, :, perm]

    edges = jnp.arange(0, N_OUT + 1, RB, dtype=jnp.int32)
    bounds = jnp.searchsorted(rows, edges, side="left").astype(jnp.int32)
    bounds = jnp.pad(bounds, (0, NBOUNDS - bounds.shape[0]))

    kn = jax.random.key(42)
    rest = jnp.sum((jax.random.uniform(kn, (b, s, 10)) < 0.1)
                   .astype(jnp.float32), -1).reshape(b * s)
    rest10 = jnp.broadcast_to((rest / 10.0)[:, None], (b * s, L))

    out2 = _sc_sparse_matmul(inp2, packed, bounds,
                             bkg_weights.astype(jnp.float32), rest10)
    return out2.reshape(b, s, N_OUT)
